# baseline (speedup 1.0000x reference)
import functools

import jax
import jax.numpy as jnp
from jax import lax
from jax.experimental import pallas as pl
from jax.experimental.pallas import tpu as pltpu
from jax.experimental.pallas import tpu_sc as plsc

NODE_DIM = 128
GLOBAL_DIM = 128
MAX_PAIRS = 28
IMAX = MAX_PAIRS + 1
NUM_NODES = 400000
BSZ = 256

NC = 1
NS = 16
NW = NC * NS
NPB = BSZ // NW
PPW = NPB * MAX_PAIRS
NROWS = BSZ * MAX_PAIRS

CHUNK = 25008
NSTEP = CHUNK // 16
PLEN = 400272
SENT = NUM_NODES
STAB = 272
GTAB = 48
GCH = 112
NGC = PPW // GCH

NEG = -1000000000.0


def _sc_body(p_hbm, nf_hbm, gf_hbm, fi_hbm, fj_hbm, fg_hbm, hp_hbm,
             pbuf, starts_loc, starts_all, gbuf, gtab,
             idxi, idxj, idxg, hpb, rows, shared, sem):
    sid = lax.axis_index("s")
    cid = lax.axis_index("c")
    wid = sid * NC + cid
    iota = lax.iota(jnp.int32, 16)

    for v in range(STAB // 16):
        starts_loc[pl.ds(v * 16, 16)] = jnp.full((16,), SENT, jnp.int32)
    base = pl.multiple_of(sid * CHUNK, 16)
    pltpu.sync_copy(p_hbm.at[pl.ds(base, CHUNK + 16)], pbuf)

    def p1_step(t, carry):
        m = 8 + t * 16
        cur = pbuf[pl.ds(m, 16)]
        prev = pbuf[pl.ds(m - 1, 16)]
        bval = cur >> 1
        bnd = bval != (prev >> 1)
        nvec = base + (m - 8) + iota
        plsc.store_scatter(starts_loc, [bval], nvec, mask=bnd)
        return carry

    lax.fori_loop(0, NSTEP, p1_step, 0)

    pltpu.sync_copy(starts_loc, shared.at[pl.ds(sid * STAB, STAB)])
    plsc.subcore_barrier()
    pltpu.sync_copy(shared, starts_all)
    acc = jnp.full((16,), SENT, jnp.int32)
    for r in range(NS):
        acc = jnp.minimum(acc, starts_all[pl.ds(r * STAB + wid * NPB, 16)])

    for bl in range(NPB):
        b = wid * NPB + bl
        s = jnp.min(jnp.where(iota == bl, acc, SENT))

        for v in range(GTAB // 16):
            gtab[pl.ds(v * 16, 16)] = jnp.zeros((16,), jnp.int32)

        w0 = ((s + 8) >> 4) << 4

        def win_step(t, carry):
            cnt, ended, wstart = carry
            vals = gbuf[pl.ds(t * 16, 16)]
            pidx = wstart + t * 16 + iota
            bval = vals >> 1
            gmv = vals & 1
            after = pidx >= (s + 8)
            inseg = after & (bval == b)
            take = inseg & (gmv == 1)
            pc = plsc.cumsum(take.astype(jnp.int32))
            sel = take & ((cnt + pc) <= IMAX)
            plsc.store_compressed(gtab.at[pl.ds(cnt, 16)], pidx - 8, mask=sel)
            cnt = cnt + jnp.sum(sel.astype(jnp.int32))
            ended = jnp.maximum(ended, jnp.max((after & (bval != b)).astype(jnp.int32)))
            return cnt, ended, wstart

        def win_body(carry):
            cnt, ended, wstart = carry
            pltpu.sync_copy(p_hbm.at[pl.ds(pl.multiple_of(wstart, 16), 256)], gbuf)
            cnt, ended, _ = lax.fori_loop(0, 16, win_step, (cnt, ended, wstart))
            return cnt, ended, wstart + 256

        def win_cond(carry):
            cnt, ended, _ = carry
            return (cnt < IMAX) & (ended == 0)

        cc, _, _ = lax.while_loop(win_cond, win_body, (jnp.int32(0), jnp.int32(0), w0))

        tpairs = jnp.minimum(cc * (cc - 1) // 2, MAX_PAIRS)
        for h in range(2):
            kvec = h * 16 + iota
            kmask = kvec < MAX_PAIRS

            def ik_step(ii, ik):
                off = ii * cc - ii * (ii + 1) // 2
                term = (kvec >= off) & (ii <= cc - 2)
                return ik + term.astype(jnp.int32)

            i_k = lax.fori_loop(1, MAX_PAIRS, ik_step, jnp.zeros((16,), jnp.int32))
            offi = i_k * cc - i_k * (i_k + 1) // 2
            j_k = jnp.clip(kvec - offi + i_k + 1, 0, GTAB - 1)
            hp = (kvec < tpairs).astype(jnp.int32)
            gi = plsc.load_gather(gtab, [i_k])
            gj = plsc.load_gather(gtab, [j_k])
            bg = jnp.zeros((16,), jnp.int32) + b
            off0 = bl * MAX_PAIRS + h * 16
            if h == 0:
                idxi[pl.ds(off0, 16)] = gi
                idxj[pl.ds(off0, 16)] = gj
                idxg[pl.ds(off0, 16)] = bg
                hpb[pl.ds(off0, 16)] = hp
            else:
                plsc.store_compressed(idxi.at[pl.ds(off0, 16)], gi, mask=kmask)
                plsc.store_compressed(idxj.at[pl.ds(off0, 16)], gj, mask=kmask)
                plsc.store_compressed(idxg.at[pl.ds(off0, 16)], bg, mask=kmask)
                plsc.store_compressed(hpb.at[pl.ds(off0, 16)], hp, mask=kmask)

    obase = wid * PPW
    for tab, idxb, outh in ((nf_hbm, idxi, fi_hbm), (nf_hbm, idxj, fj_hbm),
                            (gf_hbm, idxg, fg_hbm)):
        for t in range(NGC):
            pltpu.async_copy(tab.at[idxb.at[pl.ds(t * GCH, GCH)]], rows, sem).wait()
            pltpu.sync_copy(rows, outh.at[pl.ds(obase + t * GCH, GCH)])
    pltpu.sync_copy(hpb.at[pl.ds(0, PPW)], hp_hbm.at[pl.ds(obase, PPW)])


_sc_gather = functools.partial(
    pl.kernel,
    out_type=(
        jax.ShapeDtypeStruct((NROWS, NODE_DIM), jnp.float32),
        jax.ShapeDtypeStruct((NROWS, NODE_DIM), jnp.float32),
        jax.ShapeDtypeStruct((NROWS, GLOBAL_DIM), jnp.float32),
        jax.ShapeDtypeStruct((NROWS,), jnp.int32),
    ),
    mesh=plsc.VectorSubcoreMesh(core_axis_name="c", subcore_axis_name="s",
                                num_cores=NC),
    scratch_types=(
        pltpu.VMEM((CHUNK + 16,), jnp.int32),
        pltpu.VMEM((STAB,), jnp.int32),
        pltpu.VMEM((NS * STAB,), jnp.int32),
        pltpu.VMEM((256,), jnp.int32),
        pltpu.VMEM((GTAB,), jnp.int32),
        pltpu.VMEM((PPW + 8,), jnp.int32),
        pltpu.VMEM((PPW + 8,), jnp.int32),
        pltpu.VMEM((PPW + 8,), jnp.int32),
        pltpu.VMEM((PPW + 8,), jnp.int32),
        pltpu.VMEM((GCH, NODE_DIM), jnp.float32),
        pltpu.VMEM_SHARED((NS * STAB,), jnp.int32),
        pltpu.SemaphoreType.DMA,
    ),
    compiler_params=pltpu.CompilerParams(needs_layout_passes=False),
)(_sc_body)


def _mlp_body(fi, fj, fg, w1a, w1b, w1g, b1, w2, b2, w3, b3, hp, vm, out):
    x = jnp.dot(fi[...], w1a[...], preferred_element_type=jnp.float32)
    x = x + jnp.dot(fj[...], w1b[...], preferred_element_type=jnp.float32)
    x = x + jnp.dot(fg[...], w1g[...], preferred_element_type=jnp.float32)
    h1 = jnp.maximum(x + b1[...], 0.0)
    h2 = jnp.maximum(
        jnp.dot(h1, w2[...], preferred_element_type=jnp.float32) + b2[...], 0.0)
    s = jnp.dot(h2, w3[...], preferred_element_type=jnp.float32) + b3[...]
    ok = (hp[...] > 0) & (vm[...] > 0)
    out[...] = jnp.where(ok, s, jnp.float32(NEG))


def kernel(node_features, global_features, group_mask, docking_valid_mask,
           batch, W1, b1, W2, b2, W3, b3):
    packed = (batch.astype(jnp.int32) << 1) | group_mask.astype(jnp.int32)
    p = jnp.concatenate([
        jnp.full((8,), -2, jnp.int32),
        packed,
        jnp.full((PLEN - 8 - NUM_NODES,), 2 * BSZ, jnp.int32),
    ])

    fi, fj, fg, hp = _sc_gather(p, node_features, global_features)

    w1a = W1[:NODE_DIM]
    w1b = W1[NODE_DIM:2 * NODE_DIM]
    w1g = W1[2 * NODE_DIM:]
    w3p = jnp.zeros((W3.shape[0], 128), jnp.float32).at[:, :1].set(W3)
    vm = docking_valid_mask.reshape(NROWS, 1).astype(jnp.int32)

    out = pl.pallas_call(
        _mlp_body,
        out_shape=jax.ShapeDtypeStruct((NROWS, 128), jnp.float32),
    )(fi, fj, fg, w1a, w1b, w1g, b1.reshape(1, -1), W2, b2.reshape(1, -1),
      w3p, b3.reshape(1, 1), hp.reshape(NROWS, 1), vm)

    return out[:, 0].reshape(BSZ, MAX_PAIRS)


# speedup vs baseline: 1.2220x; 1.2220x over previous
import functools

import jax
import jax.numpy as jnp
from jax import lax
from jax.experimental import pallas as pl
from jax.experimental.pallas import tpu as pltpu
from jax.experimental.pallas import tpu_sc as plsc

NODE_DIM = 128
GLOBAL_DIM = 128
MAX_PAIRS = 28
IMAX = MAX_PAIRS + 1
NUM_NODES = 400000
BSZ = 256

NC = 2
NS = 16
NW = NC * NS
NPB = BSZ // NW
PPW = NPB * MAX_PAIRS
NROWS = BSZ * MAX_PAIRS

CHUNK = 25008
NSTEP = CHUNK // 16
PLEN = 400272
SENT = NUM_NODES
STAB = 272
GTAB = 48
GCH = 112
NGC = PPW // GCH

NEG = -1000000000.0


def _sc_body(p_hbm, nf_hbm, gf_hbm, fi_hbm, fj_hbm, fg_hbm, hp_hbm,
             pbuf, starts_loc, starts_all, gbuf, gtab,
             idxi, idxj, idxg, hpb, rows, shared, sem):
    sid = lax.axis_index("s")
    cid = lax.axis_index("c")
    wid = sid * NC + cid
    iota = lax.iota(jnp.int32, 16)

    scope1 = jax.named_scope("p1_scan")
    scope1.__enter__()
    for v in range(STAB // 16):
        starts_loc[pl.ds(v * 16, 16)] = jnp.full((16,), SENT, jnp.int32)
    base = pl.multiple_of(sid * CHUNK, 16)
    pltpu.sync_copy(p_hbm.at[pl.ds(base, CHUNK + 16)], pbuf)

    def p1_step(t, carry):
        m = 8 + t * 16
        cur = pbuf[pl.ds(m, 16)]
        prev = pbuf[pl.ds(m - 1, 16)]
        bval = cur >> 1
        bnd = bval != (prev >> 1)
        nvec = base + (m - 8) + iota
        plsc.store_scatter(starts_loc, [bval], nvec, mask=bnd)
        return carry

    lax.fori_loop(0, NSTEP, p1_step, 0)
    scope1.__exit__(None, None, None)
    scope2 = jax.named_scope("p2_share")
    scope2.__enter__()

    pltpu.sync_copy(starts_loc, shared.at[pl.ds(sid * STAB, STAB)])
    plsc.subcore_barrier()
    pltpu.sync_copy(shared, starts_all)
    acc = jnp.full((16,), SENT, jnp.int32)
    for r in range(NS):
        acc = jnp.minimum(acc, starts_all[pl.ds(r * STAB + wid * NPB, 16)])

    scope2.__exit__(None, None, None)
    scope3 = jax.named_scope("p3_batches")
    scope3.__enter__()
    for bl in range(NPB):
        b = wid * NPB + bl
        s = jnp.min(jnp.where(iota == bl, acc, SENT))

        for v in range(GTAB // 16):
            gtab[pl.ds(v * 16, 16)] = jnp.zeros((16,), jnp.int32)

        w0 = ((s + 8) >> 4) << 4

        def win_step(t, carry):
            cnt, ended, wstart = carry
            vals = gbuf[pl.ds(t * 16, 16)]
            pidx = wstart + t * 16 + iota
            bval = vals >> 1
            gmv = vals & 1
            after = pidx >= (s + 8)
            inseg = after & (bval == b)
            take = inseg & (gmv == 1)
            pc = plsc.cumsum(take.astype(jnp.int32))
            sel = take & ((cnt + pc) <= IMAX)
            plsc.store_compressed(gtab.at[pl.ds(cnt, 16)], pidx - 8, mask=sel)
            cnt = cnt + jnp.sum(sel.astype(jnp.int32))
            ended = jnp.maximum(ended, jnp.max((after & (bval != b)).astype(jnp.int32)))
            return cnt, ended, wstart

        def win_body(carry):
            cnt, ended, wstart = carry
            pltpu.sync_copy(p_hbm.at[pl.ds(pl.multiple_of(wstart, 16), 256)], gbuf)
            cnt, ended, _ = lax.fori_loop(0, 16, win_step, (cnt, ended, wstart))
            return cnt, ended, wstart + 256

        def win_cond(carry):
            cnt, ended, _ = carry
            return (cnt < IMAX) & (ended == 0)

        cc, _, _ = lax.while_loop(win_cond, win_body, (jnp.int32(0), jnp.int32(0), w0))

        tpairs = jnp.minimum(cc * (cc - 1) // 2, MAX_PAIRS)
        for h in range(2):
            kvec = h * 16 + iota
            kmask = kvec < MAX_PAIRS

            def ik_step(ii, ik):
                off = ii * cc - ii * (ii + 1) // 2
                term = (kvec >= off) & (ii <= cc - 2)
                return ik + term.astype(jnp.int32)

            i_k = lax.fori_loop(1, MAX_PAIRS, ik_step, jnp.zeros((16,), jnp.int32))
            offi = i_k * cc - i_k * (i_k + 1) // 2
            j_k = jnp.clip(kvec - offi + i_k + 1, 0, GTAB - 1)
            hp = (kvec < tpairs).astype(jnp.int32)
            gi = plsc.load_gather(gtab, [i_k])
            gj = plsc.load_gather(gtab, [j_k])
            bg = jnp.zeros((16,), jnp.int32) + b
            off0 = bl * MAX_PAIRS + h * 16
            if h == 0:
                idxi[pl.ds(off0, 16)] = gi
                idxj[pl.ds(off0, 16)] = gj
                idxg[pl.ds(off0, 16)] = bg
                hpb[pl.ds(off0, 16)] = hp
            else:
                plsc.store_compressed(idxi.at[pl.ds(off0, 16)], gi, mask=kmask)
                plsc.store_compressed(idxj.at[pl.ds(off0, 16)], gj, mask=kmask)
                plsc.store_compressed(idxg.at[pl.ds(off0, 16)], bg, mask=kmask)
                plsc.store_compressed(hpb.at[pl.ds(off0, 16)], hp, mask=kmask)

    scope3.__exit__(None, None, None)
    scope4 = jax.named_scope("p4_gather")
    scope4.__enter__()
    obase = wid * PPW
    for tab, idxb, outh in ((nf_hbm, idxi, fi_hbm), (nf_hbm, idxj, fj_hbm),
                            (gf_hbm, idxg, fg_hbm)):
        for t in range(NGC):
            pltpu.async_copy(tab.at[idxb.at[pl.ds(t * GCH, GCH)]], rows, sem).wait()
            pltpu.sync_copy(rows, outh.at[pl.ds(obase + t * GCH, GCH)])
    pltpu.sync_copy(hpb.at[pl.ds(0, PPW)], hp_hbm.at[pl.ds(obase, PPW)])
    scope4.__exit__(None, None, None)


_sc_gather = functools.partial(
    pl.kernel,
    out_type=(
        jax.ShapeDtypeStruct((NROWS, NODE_DIM), jnp.float32),
        jax.ShapeDtypeStruct((NROWS, NODE_DIM), jnp.float32),
        jax.ShapeDtypeStruct((NROWS, GLOBAL_DIM), jnp.float32),
        jax.ShapeDtypeStruct((NROWS,), jnp.int32),
    ),
    mesh=plsc.VectorSubcoreMesh(core_axis_name="c", subcore_axis_name="s",
                                num_cores=NC),
    scratch_types=(
        pltpu.VMEM((CHUNK + 16,), jnp.int32),
        pltpu.VMEM((STAB,), jnp.int32),
        pltpu.VMEM((NS * STAB,), jnp.int32),
        pltpu.VMEM((256,), jnp.int32),
        pltpu.VMEM((GTAB,), jnp.int32),
        pltpu.VMEM((PPW + 8,), jnp.int32),
        pltpu.VMEM((PPW + 8,), jnp.int32),
        pltpu.VMEM((PPW + 8,), jnp.int32),
        pltpu.VMEM((PPW + 8,), jnp.int32),
        pltpu.VMEM((GCH, NODE_DIM), jnp.float32),
        pltpu.VMEM_SHARED((NS * STAB,), jnp.int32),
        pltpu.SemaphoreType.DMA,
    ),
    compiler_params=pltpu.CompilerParams(needs_layout_passes=False),
)(_sc_body)


def _mlp_body(fi, fj, fg, w1a, w1b, w1g, b1, w2, b2, w3, b3, hp, vm, out):
    x = jnp.dot(fi[...], w1a[...], preferred_element_type=jnp.float32)
    x = x + jnp.dot(fj[...], w1b[...], preferred_element_type=jnp.float32)
    x = x + jnp.dot(fg[...], w1g[...], preferred_element_type=jnp.float32)
    h1 = jnp.maximum(x + b1[...], 0.0)
    h2 = jnp.maximum(
        jnp.dot(h1, w2[...], preferred_element_type=jnp.float32) + b2[...], 0.0)
    s = jnp.dot(h2, w3[...], preferred_element_type=jnp.float32) + b3[...]
    ok = (hp[...] > 0) & (vm[...] > 0)
    out[...] = jnp.where(ok, s, jnp.float32(NEG))


def kernel(node_features, global_features, group_mask, docking_valid_mask,
           batch, W1, b1, W2, b2, W3, b3):
    packed = (batch.astype(jnp.int32) << 1) | group_mask.astype(jnp.int32)
    p = jnp.concatenate([
        jnp.full((8,), -2, jnp.int32),
        packed,
        jnp.full((PLEN - 8 - NUM_NODES,), 2 * BSZ, jnp.int32),
    ])

    fi, fj, fg, hp = _sc_gather(p, node_features, global_features)

    w1a = W1[:NODE_DIM]
    w1b = W1[NODE_DIM:2 * NODE_DIM]
    w1g = W1[2 * NODE_DIM:]
    w3p = jnp.zeros((W3.shape[0], 128), jnp.float32).at[:, :1].set(W3)
    vm = docking_valid_mask.reshape(NROWS, 1).astype(jnp.int32)

    out = pl.pallas_call(
        _mlp_body,
        out_shape=jax.ShapeDtypeStruct((NROWS, 128), jnp.float32),
    )(fi, fj, fg, w1a, w1b, w1g, b1.reshape(1, -1), W2, b2.reshape(1, -1),
      w3p, b3.reshape(1, 1), hp.reshape(NROWS, 1), vm)

    return out[:, 0].reshape(BSZ, MAX_PAIRS)


# speedup vs baseline: 1.3395x; 1.0961x over previous
import functools

import jax
import jax.numpy as jnp
from jax import lax
from jax.experimental import pallas as pl
from jax.experimental.pallas import tpu as pltpu
from jax.experimental.pallas import tpu_sc as plsc

NODE_DIM = 128
GLOBAL_DIM = 128
MAX_PAIRS = 28
IMAX = MAX_PAIRS + 1
NUM_NODES = 400000
BSZ = 256

NC = 2
NS = 16
NW = NC * NS
NPB = BSZ // NW
PPW = NPB * MAX_PAIRS
NROWS = BSZ * MAX_PAIRS

CHUNK = 25008
NSTEP = CHUNK // 16
PLEN = 400272
SENT = NUM_NODES
STAB = 272
GTAB = 48
GCH = 112
NGC = PPW // GCH

NEG = -1000000000.0


def _sc_body(p_hbm, nf_hbm, gf_hbm, fi_hbm, fj_hbm, fg_hbm, hp_hbm,
             pbuf, starts_loc, starts_all, gbuf, gtab,
             idxi, idxj, idxg, hpb, rows, shared, sem):
    sid = lax.axis_index("s")
    cid = lax.axis_index("c")
    wid = sid * NC + cid
    iota = lax.iota(jnp.int32, 16)

    scope1 = jax.named_scope("p1_scan")
    scope1.__enter__()
    for v in range(STAB // 16):
        starts_loc[pl.ds(v * 16, 16)] = jnp.full((16,), SENT, jnp.int32)
    base = pl.multiple_of(sid * CHUNK, 16)
    pltpu.sync_copy(p_hbm.at[pl.ds(base, CHUNK + 16)], pbuf)

    def p1_step(t, carry):
        m = 8 + t * 16
        cur = pbuf[pl.ds(m, 16)]
        prev = pbuf[pl.ds(m - 1, 16)]
        bval = cur >> 1
        bnd = bval != (prev >> 1)
        nvec = base + (m - 8) + iota
        plsc.store_scatter(starts_loc, [bval], nvec, mask=bnd)
        return carry

    lax.fori_loop(0, NSTEP, p1_step, 0)
    scope1.__exit__(None, None, None)
    scope2 = jax.named_scope("p2_share")
    scope2.__enter__()

    pltpu.sync_copy(starts_loc, shared.at[pl.ds(sid * STAB, STAB)])
    plsc.subcore_barrier()
    pltpu.sync_copy(shared, starts_all)
    acc = jnp.full((16,), SENT, jnp.int32)
    for r in range(NS):
        acc = jnp.minimum(acc, starts_all[pl.ds(r * STAB + wid * NPB, 16)])

    scope2.__exit__(None, None, None)
    scope3 = jax.named_scope("p3_batches")
    scope3.__enter__()
    for bl in range(NPB):
        b = wid * NPB + bl
        s = jnp.min(jnp.where(iota == bl, acc, SENT))

        for v in range(GTAB // 16):
            gtab[pl.ds(v * 16, 16)] = jnp.zeros((16,), jnp.int32)

        w0 = ((s + 8) >> 4) << 4

        def win_step(t, carry):
            cnt, ended, wstart = carry
            vals = gbuf[pl.ds(t * 16, 16)]
            pidx = wstart + t * 16 + iota
            bval = vals >> 1
            gmv = vals & 1
            after = pidx >= (s + 8)
            inseg = after & (bval == b)
            take = inseg & (gmv == 1)
            pc = plsc.cumsum(take.astype(jnp.int32))
            sel = take & ((cnt + pc) <= IMAX)
            plsc.store_compressed(gtab.at[pl.ds(cnt, 16)], pidx - 8, mask=sel)
            cnt = cnt + jnp.sum(sel.astype(jnp.int32))
            ended = jnp.maximum(ended, jnp.max((after & (bval != b)).astype(jnp.int32)))
            return cnt, ended, wstart

        def win_body(carry):
            cnt, ended, wstart = carry
            pltpu.sync_copy(p_hbm.at[pl.ds(pl.multiple_of(wstart, 16), 256)], gbuf)
            cnt, ended, _ = lax.fori_loop(0, 16, win_step, (cnt, ended, wstart))
            return cnt, ended, wstart + 256

        def win_cond(carry):
            cnt, ended, _ = carry
            return (cnt < IMAX) & (ended == 0)

        cc, _, _ = lax.while_loop(win_cond, win_body, (jnp.int32(0), jnp.int32(0), w0))

        tpairs = jnp.minimum(cc * (cc - 1) // 2, MAX_PAIRS)
        for h in range(2):
            kvec = h * 16 + iota
            kmask = kvec < MAX_PAIRS

            def ik_step(ii, ik):
                off = ii * cc - ii * (ii + 1) // 2
                term = (kvec >= off) & (ii <= cc - 2)
                return ik + term.astype(jnp.int32)

            i_k = lax.fori_loop(1, MAX_PAIRS, ik_step, jnp.zeros((16,), jnp.int32))
            offi = i_k * cc - i_k * (i_k + 1) // 2
            j_k = jnp.clip(kvec - offi + i_k + 1, 0, GTAB - 1)
            hp = (kvec < tpairs).astype(jnp.int32)
            gi = plsc.load_gather(gtab, [i_k])
            gj = plsc.load_gather(gtab, [j_k])
            bg = jnp.zeros((16,), jnp.int32) + b
            off0 = bl * MAX_PAIRS + h * 16
            if h == 0:
                idxi[pl.ds(off0, 16)] = gi
                idxj[pl.ds(off0, 16)] = gj
                idxg[pl.ds(off0, 16)] = bg
                hpb[pl.ds(off0, 16)] = hp
            else:
                plsc.store_compressed(idxi.at[pl.ds(off0, 16)], gi, mask=kmask)
                plsc.store_compressed(idxj.at[pl.ds(off0, 16)], gj, mask=kmask)
                plsc.store_compressed(idxg.at[pl.ds(off0, 16)], bg, mask=kmask)
                plsc.store_compressed(hpb.at[pl.ds(off0, 16)], hp, mask=kmask)

    scope3.__exit__(None, None, None)
    scope4 = jax.named_scope("p4_gather")
    scope4.__enter__()
    obase = wid * PPW
    jobs = []
    r = 0
    for tab, idxb, outh in ((nf_hbm, idxi, fi_hbm), (nf_hbm, idxj, fj_hbm),
                            (gf_hbm, idxg, fg_hbm)):
        for t in range(NGC):
            jobs.append((tab, idxb, outh, t, r))
            r += 1
    descs = [pltpu.async_copy(tab.at[idxb.at[pl.ds(t * GCH, GCH)]],
                              rows.at[r], sem)
             for tab, idxb, outh, t, r in jobs]
    for d in descs:
        d.wait()
    outs = [pltpu.async_copy(rows.at[r], outh.at[pl.ds(obase + t * GCH, GCH)],
                             sem)
            for tab, idxb, outh, t, r in jobs]
    for d in outs:
        d.wait()
    pltpu.sync_copy(hpb.at[pl.ds(0, PPW)], hp_hbm.at[pl.ds(obase, PPW)])
    scope4.__exit__(None, None, None)


_sc_gather = functools.partial(
    pl.kernel,
    out_type=(
        jax.ShapeDtypeStruct((NROWS, NODE_DIM), jnp.float32),
        jax.ShapeDtypeStruct((NROWS, NODE_DIM), jnp.float32),
        jax.ShapeDtypeStruct((NROWS, GLOBAL_DIM), jnp.float32),
        jax.ShapeDtypeStruct((NROWS,), jnp.int32),
    ),
    mesh=plsc.VectorSubcoreMesh(core_axis_name="c", subcore_axis_name="s",
                                num_cores=NC),
    scratch_types=(
        pltpu.VMEM((CHUNK + 16,), jnp.int32),
        pltpu.VMEM((STAB,), jnp.int32),
        pltpu.VMEM((NS * STAB,), jnp.int32),
        pltpu.VMEM((256,), jnp.int32),
        pltpu.VMEM((GTAB,), jnp.int32),
        pltpu.VMEM((PPW + 8,), jnp.int32),
        pltpu.VMEM((PPW + 8,), jnp.int32),
        pltpu.VMEM((PPW + 8,), jnp.int32),
        pltpu.VMEM((PPW + 8,), jnp.int32),
        pltpu.VMEM((3 * NGC, GCH, NODE_DIM), jnp.float32),
        pltpu.VMEM_SHARED((NS * STAB,), jnp.int32),
        pltpu.SemaphoreType.DMA,
    ),
    compiler_params=pltpu.CompilerParams(needs_layout_passes=False),
)(_sc_body)


def _mlp_body(fi, fj, fg, w1a, w1b, w1g, b1, w2, b2, w3, b3, hp, vm, out):
    bf = jnp.bfloat16
    x = jnp.dot(fi[...].astype(bf), w1a[...].astype(bf),
                preferred_element_type=jnp.float32)
    x = x + jnp.dot(fj[...].astype(bf), w1b[...].astype(bf),
                    preferred_element_type=jnp.float32)
    x = x + jnp.dot(fg[...].astype(bf), w1g[...].astype(bf),
                    preferred_element_type=jnp.float32)
    h1 = jnp.maximum(x + b1[...], 0.0)
    h2 = jnp.maximum(
        jnp.dot(h1, w2[...], preferred_element_type=jnp.float32) + b2[...], 0.0)
    s = jnp.dot(h2, w3[...], preferred_element_type=jnp.float32) + b3[...]
    ok = (hp[...] > 0) & (vm[...] > 0)
    out[...] = jnp.where(ok, s, jnp.float32(NEG))


def kernel(node_features, global_features, group_mask, docking_valid_mask,
           batch, W1, b1, W2, b2, W3, b3):
    packed = (batch.astype(jnp.int32) << 1) | group_mask.astype(jnp.int32)
    p = jnp.concatenate([
        jnp.full((8,), -2, jnp.int32),
        packed,
        jnp.full((PLEN - 8 - NUM_NODES,), 2 * BSZ, jnp.int32),
    ])

    fi, fj, fg, hp = _sc_gather(p, node_features, global_features)

    w1a = W1[:NODE_DIM]
    w1b = W1[NODE_DIM:2 * NODE_DIM]
    w1g = W1[2 * NODE_DIM:]
    vm = docking_valid_mask.reshape(NROWS, 1).astype(jnp.int32)

    out = pl.pallas_call(
        _mlp_body,
        out_shape=jax.ShapeDtypeStruct((NROWS, 1), jnp.float32),
    )(fi, fj, fg, w1a, w1b, w1g, b1.reshape(1, -1), W2, b2.reshape(1, -1),
      W3, b3.reshape(1, 1), hp.reshape(NROWS, 1), vm)

    return out.reshape(BSZ, MAX_PAIRS)


# speedup vs baseline: 1.4898x; 1.1122x over previous
import functools

import jax
import jax.numpy as jnp
from jax import lax
from jax.experimental import pallas as pl
from jax.experimental.pallas import tpu as pltpu
from jax.experimental.pallas import tpu_sc as plsc

NODE_DIM = 128
GLOBAL_DIM = 128
MAX_PAIRS = 28
IMAX = MAX_PAIRS + 1
NUM_NODES = 400000
BSZ = 256

NC = 2
NS = 16
NW = NC * NS
NPB = BSZ // NW
PPW = NPB * MAX_PAIRS
NROWS = BSZ * MAX_PAIRS

CHUNK = 25008
NSTEP = CHUNK // 16
LCHUNK = NUM_NODES - (NS - 1) * CHUNK
LSTEP = LCHUNK // 16
SENT = NUM_NODES
STAB = 272
GTAB = 48
GCH = 112
NGC = PPW // GCH

NEG = -1000000000.0


def _sc_body(b_hbm, g_hbm, nf_hbm, gf_hbm, fi_hbm, fj_hbm, fg_hbm, hp_hbm,
             pbuf, starts_loc, starts_all, bwin, gwin, gtab,
             idxi, idxj, idxg, hpb, rows, shared, sem):
    sid = lax.axis_index("s")
    cid = lax.axis_index("c")
    wid = sid * NC + cid
    iota = lax.iota(jnp.int32, 16)

    scope1 = jax.named_scope("p1_scan")
    scope1.__enter__()
    for v in range(STAB // 16):
        starts_loc[pl.ds(v * 16, 16)] = jnp.full((16,), SENT, jnp.int32)
    base = pl.multiple_of(sid * CHUNK, 16)

    @pl.when(sid == 0)
    def _():
        pbuf[pl.ds(0, 16)] = jnp.full((16,), -1, jnp.int32)
        pltpu.sync_copy(b_hbm.at[pl.ds(0, CHUNK)], pbuf.at[pl.ds(8, CHUNK)])

    @pl.when((sid > 0) & (sid < NS - 1))
    def _():
        pltpu.sync_copy(b_hbm.at[pl.ds(base - 8, CHUNK + 8)],
                        pbuf.at[pl.ds(0, CHUNK + 8)])

    @pl.when(sid == NS - 1)
    def _():
        pltpu.sync_copy(b_hbm.at[pl.ds(base - 8, LCHUNK + 8)],
                        pbuf.at[pl.ds(0, LCHUNK + 8)])

    def p1_step(t, carry):
        m = 8 + t * 16
        cur = pbuf[pl.ds(m, 16)]
        prev = pbuf[pl.ds(m - 1, 16)]
        bnd = cur != prev
        nvec = base + (m - 8) + iota
        plsc.store_scatter(starts_loc, [cur], nvec, mask=bnd)
        return carry

    nstep = jnp.where(sid == NS - 1, LSTEP, NSTEP)
    lax.fori_loop(0, nstep, p1_step, 0)
    scope1.__exit__(None, None, None)
    scope2 = jax.named_scope("p2_share")
    scope2.__enter__()

    pltpu.sync_copy(starts_loc, shared.at[pl.ds(sid * STAB, STAB)])
    plsc.subcore_barrier()
    pltpu.sync_copy(shared, starts_all)
    acc = jnp.full((16,), SENT, jnp.int32)
    for r in range(NS):
        acc = jnp.minimum(acc, starts_all[pl.ds(r * STAB + wid * NPB, 16)])

    scope2.__exit__(None, None, None)
    scope3 = jax.named_scope("p3_batches")
    scope3.__enter__()
    for bl in range(NPB):
        b = wid * NPB + bl
        s = jnp.min(jnp.where(iota == bl, acc, SENT))

        for v in range(GTAB // 16):
            gtab[pl.ds(v * 16, 16)] = jnp.zeros((16,), jnp.int32)

        w0 = (s >> 4) << 4

        def win_body(carry):
            cnt, ended, wstart = carry
            off = pl.multiple_of(jnp.minimum(wstart, NUM_NODES - 256), 16)
            shift = wstart - off

            d1 = pltpu.async_copy(b_hbm.at[pl.ds(off, 256)], bwin, sem)
            d2 = pltpu.async_copy(g_hbm.at[pl.ds(off, 256)], gwin, sem)
            d1.wait()
            d2.wait()

            def win_step(t, carry2):
                cnt, ended = carry2
                bv = bwin[pl.ds(shift + t * 16, 16)]
                gv = gwin[pl.ds(shift + t * 16, 16)]
                pidx = wstart + t * 16 + iota
                after = pidx >= s
                inseg = after & (bv == b)
                take = inseg & (gv == 1)
                pc = plsc.cumsum(take.astype(jnp.int32))
                sel = take & ((cnt + pc) <= IMAX)
                plsc.store_compressed(gtab.at[pl.ds(cnt, 16)], pidx, mask=sel)
                cnt = cnt + jnp.sum(sel.astype(jnp.int32))
                ended = jnp.maximum(
                    ended, jnp.max((after & (bv != b)).astype(jnp.int32)))
                return cnt, ended

            nst = (jnp.int32(256) - shift) >> 4
            cnt, ended = lax.fori_loop(0, nst, win_step, (cnt, ended))
            ended = jnp.maximum(
                ended, (wstart + 256 >= NUM_NODES).astype(jnp.int32))
            return cnt, ended, wstart + 256

        def win_cond(carry):
            cnt, ended, _ = carry
            return (cnt < IMAX) & (ended == 0)

        cc, _, _ = lax.while_loop(win_cond, win_body, (jnp.int32(0), jnp.int32(0), w0))

        tpairs = jnp.minimum(cc * (cc - 1) // 2, MAX_PAIRS)
        for h in range(2):
            kvec = h * 16 + iota
            kmask = kvec < MAX_PAIRS

            def ik_step(ii, ik):
                off = ii * cc - ii * (ii + 1) // 2
                term = (kvec >= off) & (ii <= cc - 2)
                return ik + term.astype(jnp.int32)

            i_k = lax.fori_loop(1, MAX_PAIRS, ik_step, jnp.zeros((16,), jnp.int32))
            offi = i_k * cc - i_k * (i_k + 1) // 2
            j_k = jnp.clip(kvec - offi + i_k + 1, 0, GTAB - 1)
            hp = (kvec < tpairs).astype(jnp.int32)
            gi = plsc.load_gather(gtab, [i_k])
            gj = plsc.load_gather(gtab, [j_k])
            bg = jnp.zeros((16,), jnp.int32) + b
            off0 = bl * MAX_PAIRS + h * 16
            if h == 0:
                idxi[pl.ds(off0, 16)] = gi
                idxj[pl.ds(off0, 16)] = gj
                idxg[pl.ds(off0, 16)] = bg
                hpb[pl.ds(off0, 16)] = hp
            else:
                plsc.store_compressed(idxi.at[pl.ds(off0, 16)], gi, mask=kmask)
                plsc.store_compressed(idxj.at[pl.ds(off0, 16)], gj, mask=kmask)
                plsc.store_compressed(idxg.at[pl.ds(off0, 16)], bg, mask=kmask)
                plsc.store_compressed(hpb.at[pl.ds(off0, 16)], hp, mask=kmask)

    scope3.__exit__(None, None, None)
    scope4 = jax.named_scope("p4_gather")
    scope4.__enter__()
    obase = wid * PPW
    jobs = []
    r = 0
    for tab, idxb, outh in ((nf_hbm, idxi, fi_hbm), (nf_hbm, idxj, fj_hbm),
                            (gf_hbm, idxg, fg_hbm)):
        for t in range(NGC):
            jobs.append((tab, idxb, outh, t, r))
            r += 1
    descs = [pltpu.async_copy(tab.at[idxb.at[pl.ds(t * GCH, GCH)]],
                              rows.at[r], sem)
             for tab, idxb, outh, t, r in jobs]
    for d in descs:
        d.wait()
    outs = [pltpu.async_copy(rows.at[r], outh.at[pl.ds(obase + t * GCH, GCH)],
                             sem)
            for tab, idxb, outh, t, r in jobs]
    for d in outs:
        d.wait()
    pltpu.sync_copy(hpb.at[pl.ds(0, PPW)], hp_hbm.at[pl.ds(obase, PPW)])
    scope4.__exit__(None, None, None)


_sc_gather = functools.partial(
    pl.kernel,
    out_type=(
        jax.ShapeDtypeStruct((NROWS, NODE_DIM), jnp.float32),
        jax.ShapeDtypeStruct((NROWS, NODE_DIM), jnp.float32),
        jax.ShapeDtypeStruct((NROWS, GLOBAL_DIM), jnp.float32),
        jax.ShapeDtypeStruct((NROWS,), jnp.int32),
    ),
    mesh=plsc.VectorSubcoreMesh(core_axis_name="c", subcore_axis_name="s",
                                num_cores=NC),
    scratch_types=(
        pltpu.VMEM((CHUNK + 16,), jnp.int32),
        pltpu.VMEM((STAB,), jnp.int32),
        pltpu.VMEM((NS * STAB,), jnp.int32),
        pltpu.VMEM((256,), jnp.int32),
        pltpu.VMEM((256,), jnp.int32),
        pltpu.VMEM((GTAB,), jnp.int32),
        pltpu.VMEM((PPW + 8,), jnp.int32),
        pltpu.VMEM((PPW + 8,), jnp.int32),
        pltpu.VMEM((PPW + 8,), jnp.int32),
        pltpu.VMEM((PPW + 8,), jnp.int32),
        pltpu.VMEM((3 * NGC, GCH, NODE_DIM), jnp.float32),
        pltpu.VMEM_SHARED((NS * STAB,), jnp.int32),
        pltpu.SemaphoreType.DMA,
    ),
    compiler_params=pltpu.CompilerParams(needs_layout_passes=False),
)(_sc_body)


def _mlp_body(fi, fj, fg, w1a, w1b, w1g, b1, w2, b2, w3, b3, hp, vm, out):
    bf = jnp.bfloat16
    x = jnp.dot(fi[...].astype(bf), w1a[...].astype(bf),
                preferred_element_type=jnp.float32)
    x = x + jnp.dot(fj[...].astype(bf), w1b[...].astype(bf),
                    preferred_element_type=jnp.float32)
    x = x + jnp.dot(fg[...].astype(bf), w1g[...].astype(bf),
                    preferred_element_type=jnp.float32)
    h1 = jnp.maximum(x + b1[...], 0.0)
    h2 = jnp.maximum(
        jnp.dot(h1, w2[...], preferred_element_type=jnp.float32) + b2[...], 0.0)
    s = jnp.dot(h2, w3[...], preferred_element_type=jnp.float32) + b3[...]
    ok = (hp[...] > 0) & (vm[...] > 0)
    out[...] = jnp.where(ok, s, jnp.float32(NEG))


def kernel(node_features, global_features, group_mask, docking_valid_mask,
           batch, W1, b1, W2, b2, W3, b3):
    fi, fj, fg, hp = _sc_gather(batch.astype(jnp.int32),
                                group_mask.astype(jnp.int32),
                                node_features, global_features)

    w1a = W1[:NODE_DIM]
    w1b = W1[NODE_DIM:2 * NODE_DIM]
    w1g = W1[2 * NODE_DIM:]
    vm = docking_valid_mask.reshape(NROWS, 1).astype(jnp.int32)

    out = pl.pallas_call(
        _mlp_body,
        out_shape=jax.ShapeDtypeStruct((NROWS, 1), jnp.float32),
    )(fi, fj, fg, w1a, w1b, w1g, b1.reshape(1, -1), W2, b2.reshape(1, -1),
      W3, b3.reshape(1, 1), hp.reshape(NROWS, 1), vm)

    return out.reshape(BSZ, MAX_PAIRS)


# speedup vs baseline: 1.6703x; 1.1211x over previous
import functools

import jax
import jax.numpy as jnp
from jax import lax
from jax.experimental import pallas as pl
from jax.experimental.pallas import tpu as pltpu
from jax.experimental.pallas import tpu_sc as plsc

NODE_DIM = 128
GLOBAL_DIM = 128
MAX_PAIRS = 28
IMAX = MAX_PAIRS + 1
NUM_NODES = 400000
BSZ = 256

NC = 2
NS = 16
NW = NC * NS
NPB = BSZ // NW
PPW = NPB * MAX_PAIRS
NROWS = BSZ * MAX_PAIRS

CHUNK = 25008
NSTEP = CHUNK // 16
LCHUNK = NUM_NODES - (NS - 1) * CHUNK
LSTEP = LCHUNK // 16
SENT = NUM_NODES
STAB = 272
WL = 288
GTAB = 48
GCH = 112
NGC = PPW // GCH

NEG = -1000000000.0


def _sc_body(b_hbm, g_hbm, nf_hbm, gf_hbm, fi_hbm, fj_hbm, fg_hbm, hp_hbm,
             pbuf, starts_loc, starts_all, wl, bwin, gwin, gtab,
             idxi, idxj, idxg, hpb, rows, shared, sem):
    sid = lax.axis_index("s")
    cid = lax.axis_index("c")
    wid = sid * NC + cid
    iota = lax.iota(jnp.int32, 16)

    scope1 = jax.named_scope("p1_scan")
    scope1.__enter__()
    for v in range(STAB // 16):
        starts_loc[pl.ds(v * 16, 16)] = jnp.full((16,), SENT, jnp.int32)
    base = pl.multiple_of(sid * CHUNK, 16)

    @pl.when(sid == 0)
    def _():
        pbuf[pl.ds(0, 16)] = jnp.full((16,), -1, jnp.int32)
        pltpu.sync_copy(b_hbm.at[pl.ds(0, CHUNK)], pbuf.at[pl.ds(8, CHUNK)])

    @pl.when((sid > 0) & (sid < NS - 1))
    def _():
        pltpu.sync_copy(b_hbm.at[pl.ds(base - 8, CHUNK + 8)],
                        pbuf.at[pl.ds(0, CHUNK + 8)])

    @pl.when(sid == NS - 1)
    def _():
        pltpu.sync_copy(b_hbm.at[pl.ds(base - 8, LCHUNK + 8)],
                        pbuf.at[pl.ds(0, LCHUNK + 8)])

    nstep = jnp.where(sid == NS - 1, LSTEP, NSTEP)

    def screen_step(q, wcount):
        blockid = q * 16 + iota
        bc = jnp.minimum(blockid, nstep - 1)
        va = plsc.load_gather(pbuf, [8 + bc * 16 - 1])
        vb = plsc.load_gather(pbuf, [8 + bc * 16 + 15])
        m = (blockid < nstep) & (va != vb)
        plsc.store_compressed(wl.at[pl.ds(wcount, 16)], blockid, mask=m)
        return wcount + jnp.sum(m.astype(jnp.int32))

    for v in range(WL // 16):
        wl[pl.ds(v * 16, 16)] = jnp.zeros((16,), jnp.int32)
    wcount = lax.fori_loop(0, (NSTEP + 15) // 16, screen_step, jnp.int32(0))

    def fine_step(g, carry):
        wv = wl[pl.ds(g * 16, 16)]
        wmask = (g * 16 + iota) < wcount
        last = plsc.load_gather(pbuf, [8 + wv * 16 - 1])
        for off in range(16):
            cur = plsc.load_gather(pbuf, [8 + wv * 16 + off])
            bnd = (cur != last) & wmask
            plsc.store_scatter(starts_loc, [cur], base + wv * 16 + off,
                               mask=bnd)
            last = cur
        return carry

    lax.fori_loop(0, (wcount + 15) >> 4, fine_step, 0)
    scope1.__exit__(None, None, None)
    scope2 = jax.named_scope("p2_share")
    scope2.__enter__()

    pltpu.sync_copy(starts_loc, shared.at[pl.ds(sid * STAB, STAB)])
    plsc.subcore_barrier()
    pltpu.sync_copy(shared, starts_all)
    acc = jnp.full((16,), SENT, jnp.int32)
    for r in range(NS):
        acc = jnp.minimum(acc, starts_all[pl.ds(r * STAB + wid * NPB, 16)])

    scope2.__exit__(None, None, None)
    scope3 = jax.named_scope("p3_batches")
    scope3.__enter__()
    for bl in range(NPB):
        b = wid * NPB + bl
        s = jnp.min(jnp.where(iota == bl, acc, SENT))

        for v in range(GTAB // 16):
            gtab[pl.ds(v * 16, 16)] = jnp.zeros((16,), jnp.int32)

        w0 = (s >> 4) << 4

        def win_body(carry):
            cnt, ended, wstart = carry
            off = pl.multiple_of(jnp.minimum(wstart, NUM_NODES - 256), 16)
            shift = wstart - off

            d1 = pltpu.async_copy(b_hbm.at[pl.ds(off, 256)], bwin, sem)
            d2 = pltpu.async_copy(g_hbm.at[pl.ds(off, 256)], gwin, sem)
            d1.wait()
            d2.wait()

            def win_step(t, carry2):
                cnt, ended = carry2
                bv = bwin[pl.ds(shift + t * 16, 16)]
                gv = gwin[pl.ds(shift + t * 16, 16)]
                pidx = wstart + t * 16 + iota
                after = pidx >= s
                inseg = after & (bv == b)
                take = inseg & (gv == 1)
                pc = plsc.cumsum(take.astype(jnp.int32))
                sel = take & ((cnt + pc) <= IMAX)
                plsc.store_compressed(gtab.at[pl.ds(cnt, 16)], pidx, mask=sel)
                cnt = cnt + jnp.sum(sel.astype(jnp.int32))
                ended = jnp.maximum(
                    ended, jnp.max((after & (bv != b)).astype(jnp.int32)))
                return cnt, ended

            nst = (jnp.int32(256) - shift) >> 4
            cnt, ended = lax.fori_loop(0, nst, win_step, (cnt, ended))
            ended = jnp.maximum(
                ended, (wstart + 256 >= NUM_NODES).astype(jnp.int32))
            return cnt, ended, wstart + 256

        def win_cond(carry):
            cnt, ended, _ = carry
            return (cnt < IMAX) & (ended == 0)

        cc, _, _ = lax.while_loop(win_cond, win_body, (jnp.int32(0), jnp.int32(0), w0))

        tpairs = jnp.minimum(cc * (cc - 1) // 2, MAX_PAIRS)
        for h in range(2):
            kvec = h * 16 + iota
            kmask = kvec < MAX_PAIRS

            def ik_step(ii, ik):
                off = ii * cc - ii * (ii + 1) // 2
                term = (kvec >= off) & (ii <= cc - 2)
                return ik + term.astype(jnp.int32)

            i_k = lax.fori_loop(1, MAX_PAIRS, ik_step, jnp.zeros((16,), jnp.int32))
            offi = i_k * cc - i_k * (i_k + 1) // 2
            j_k = jnp.clip(kvec - offi + i_k + 1, 0, GTAB - 1)
            hp = (kvec < tpairs).astype(jnp.int32)
            gi = plsc.load_gather(gtab, [i_k])
            gj = plsc.load_gather(gtab, [j_k])
            bg = jnp.zeros((16,), jnp.int32) + b
            off0 = bl * MAX_PAIRS + h * 16
            if h == 0:
                idxi[pl.ds(off0, 16)] = gi
                idxj[pl.ds(off0, 16)] = gj
                idxg[pl.ds(off0, 16)] = bg
                hpb[pl.ds(off0, 16)] = hp
            else:
                plsc.store_compressed(idxi.at[pl.ds(off0, 16)], gi, mask=kmask)
                plsc.store_compressed(idxj.at[pl.ds(off0, 16)], gj, mask=kmask)
                plsc.store_compressed(idxg.at[pl.ds(off0, 16)], bg, mask=kmask)
                plsc.store_compressed(hpb.at[pl.ds(off0, 16)], hp, mask=kmask)

    scope3.__exit__(None, None, None)
    scope4 = jax.named_scope("p4_gather")
    scope4.__enter__()
    obase = wid * PPW
    jobs = []
    r = 0
    for tab, idxb, outh in ((nf_hbm, idxi, fi_hbm), (nf_hbm, idxj, fj_hbm),
                            (gf_hbm, idxg, fg_hbm)):
        for t in range(NGC):
            jobs.append((tab, idxb, outh, t, r))
            r += 1
    descs = [pltpu.async_copy(tab.at[idxb.at[pl.ds(t * GCH, GCH)]],
                              rows.at[r], sem)
             for tab, idxb, outh, t, r in jobs]
    for d in descs:
        d.wait()
    outs = [pltpu.async_copy(rows.at[r], outh.at[pl.ds(obase + t * GCH, GCH)],
                             sem)
            for tab, idxb, outh, t, r in jobs]
    for d in outs:
        d.wait()
    pltpu.sync_copy(hpb.at[pl.ds(0, PPW)], hp_hbm.at[pl.ds(obase, PPW)])
    scope4.__exit__(None, None, None)


_sc_gather = functools.partial(
    pl.kernel,
    out_type=(
        jax.ShapeDtypeStruct((NROWS, NODE_DIM), jnp.float32),
        jax.ShapeDtypeStruct((NROWS, NODE_DIM), jnp.float32),
        jax.ShapeDtypeStruct((NROWS, GLOBAL_DIM), jnp.float32),
        jax.ShapeDtypeStruct((NROWS,), jnp.int32),
    ),
    mesh=plsc.VectorSubcoreMesh(core_axis_name="c", subcore_axis_name="s",
                                num_cores=NC),
    scratch_types=(
        pltpu.VMEM((CHUNK + 16,), jnp.int32),
        pltpu.VMEM((STAB,), jnp.int32),
        pltpu.VMEM((NS * STAB,), jnp.int32),
        pltpu.VMEM((WL,), jnp.int32),
        pltpu.VMEM((256,), jnp.int32),
        pltpu.VMEM((256,), jnp.int32),
        pltpu.VMEM((GTAB,), jnp.int32),
        pltpu.VMEM((PPW + 8,), jnp.int32),
        pltpu.VMEM((PPW + 8,), jnp.int32),
        pltpu.VMEM((PPW + 8,), jnp.int32),
        pltpu.VMEM((PPW + 8,), jnp.int32),
        pltpu.VMEM((3 * NGC, GCH, NODE_DIM), jnp.float32),
        pltpu.VMEM_SHARED((NS * STAB,), jnp.int32),
        pltpu.SemaphoreType.DMA,
    ),
    compiler_params=pltpu.CompilerParams(needs_layout_passes=False),
)(_sc_body)


def _mlp_body(fi, fj, fg, w1a, w1b, w1g, b1, w2, b2, w3, b3, hp, vm, out):
    bf = jnp.bfloat16
    x = jnp.dot(fi[...].astype(bf), w1a[...].astype(bf),
                preferred_element_type=jnp.float32)
    x = x + jnp.dot(fj[...].astype(bf), w1b[...].astype(bf),
                    preferred_element_type=jnp.float32)
    x = x + jnp.dot(fg[...].astype(bf), w1g[...].astype(bf),
                    preferred_element_type=jnp.float32)
    h1 = jnp.maximum(x + b1[...], 0.0)
    h2 = jnp.maximum(
        jnp.dot(h1, w2[...], preferred_element_type=jnp.float32) + b2[...], 0.0)
    s = jnp.dot(h2, w3[...], preferred_element_type=jnp.float32) + b3[...]
    ok = (hp[...] > 0) & (vm[...] > 0)
    out[...] = jnp.where(ok, s, jnp.float32(NEG))


def kernel(node_features, global_features, group_mask, docking_valid_mask,
           batch, W1, b1, W2, b2, W3, b3):
    fi, fj, fg, hp = _sc_gather(batch.astype(jnp.int32),
                                group_mask.astype(jnp.int32),
                                node_features, global_features)

    w1a = W1[:NODE_DIM]
    w1b = W1[NODE_DIM:2 * NODE_DIM]
    w1g = W1[2 * NODE_DIM:]
    vm = docking_valid_mask.reshape(NROWS, 1).astype(jnp.int32)

    out = pl.pallas_call(
        _mlp_body,
        out_shape=jax.ShapeDtypeStruct((NROWS, 1), jnp.float32),
    )(fi, fj, fg, w1a, w1b, w1g, b1.reshape(1, -1), W2, b2.reshape(1, -1),
      W3, b3.reshape(1, 1), hp.reshape(NROWS, 1), vm)

    return out.reshape(BSZ, MAX_PAIRS)


# speedup vs baseline: 1.6739x; 1.0022x over previous
import functools

import jax
import jax.numpy as jnp
from jax import lax
from jax.experimental import pallas as pl
from jax.experimental.pallas import tpu as pltpu
from jax.experimental.pallas import tpu_sc as plsc

NODE_DIM = 128
GLOBAL_DIM = 128
MAX_PAIRS = 28
IMAX = MAX_PAIRS + 1
NUM_NODES = 400000
BSZ = 256

NC = 2
NS = 16
NW = NC * NS
NPB = BSZ // NW
PPW = NPB * MAX_PAIRS
NROWS = BSZ * MAX_PAIRS

CHUNK = 25008
NSTEP = CHUNK // 16
LCHUNK = NUM_NODES - (NS - 1) * CHUNK
LSTEP = LCHUNK // 16
SENT = NUM_NODES
STAB = 272
WL = 288
GTAB = 48
GCH = 112
NGC = PPW // GCH

NEG = -1000000000.0


def _sc_body(b_hbm, g_hbm, nf_hbm, gf_hbm, fi_hbm, fj_hbm, fg_hbm, hp_hbm,
             pbuf, starts_loc, starts_all, wl, bwin, gwin, gtab,
             idxi, idxj, idxg, hpb, rows, shared, sem):
    sid = lax.axis_index("s")
    cid = lax.axis_index("c")
    wid = sid * NC + cid
    iota = lax.iota(jnp.int32, 16)

    for v in range(STAB // 16):
        starts_loc[pl.ds(v * 16, 16)] = jnp.full((16,), SENT, jnp.int32)
    base = pl.multiple_of(sid * CHUNK, 16)

    @pl.when(sid == 0)
    def _():
        pbuf[pl.ds(0, 16)] = jnp.full((16,), -1, jnp.int32)
        pltpu.sync_copy(b_hbm.at[pl.ds(0, CHUNK)], pbuf.at[pl.ds(8, CHUNK)])

    @pl.when((sid > 0) & (sid < NS - 1))
    def _():
        pltpu.sync_copy(b_hbm.at[pl.ds(base - 8, CHUNK + 8)],
                        pbuf.at[pl.ds(0, CHUNK + 8)])

    @pl.when(sid == NS - 1)
    def _():
        pltpu.sync_copy(b_hbm.at[pl.ds(base - 8, LCHUNK + 8)],
                        pbuf.at[pl.ds(0, LCHUNK + 8)])

    nstep = jnp.where(sid == NS - 1, LSTEP, NSTEP)

    def screen_step(q, wcount):
        blockid = q * 16 + iota
        bc = jnp.minimum(blockid, nstep - 1)
        va = plsc.load_gather(pbuf, [8 + bc * 16 - 1])
        vb = plsc.load_gather(pbuf, [8 + bc * 16 + 15])
        m = (blockid < nstep) & (va != vb)
        plsc.store_compressed(wl.at[pl.ds(wcount, 16)], blockid, mask=m)
        return wcount + jnp.sum(m.astype(jnp.int32))

    for v in range(WL // 16):
        wl[pl.ds(v * 16, 16)] = jnp.zeros((16,), jnp.int32)
    wcount = lax.fori_loop(0, (NSTEP + 15) // 16, screen_step, jnp.int32(0))

    def fine_step(g, carry):
        wv = wl[pl.ds(g * 16, 16)]
        wmask = (g * 16 + iota) < wcount
        last = plsc.load_gather(pbuf, [8 + wv * 16 - 1])
        for off in range(16):
            cur = plsc.load_gather(pbuf, [8 + wv * 16 + off])
            bnd = (cur != last) & wmask
            plsc.store_scatter(starts_loc, [cur], base + wv * 16 + off,
                               mask=bnd)
            last = cur
        return carry

    lax.fori_loop(0, (wcount + 15) >> 4, fine_step, 0)

    pltpu.sync_copy(starts_loc, shared.at[pl.ds(sid * STAB, STAB)])
    plsc.subcore_barrier()
    pltpu.sync_copy(shared, starts_all)
    acc = jnp.full((16,), SENT, jnp.int32)
    for r in range(NS):
        acc = jnp.minimum(acc, starts_all[pl.ds(r * STAB + wid * NPB, 16)])

    for bl in range(NPB):
        b = wid * NPB + bl
        s = jnp.min(jnp.where(iota == bl, acc, SENT))

        for v in range(GTAB // 16):
            gtab[pl.ds(v * 16, 16)] = jnp.zeros((16,), jnp.int32)

        w0 = (s >> 4) << 4

        def win_body(carry):
            cnt, ended, wstart = carry
            off = pl.multiple_of(jnp.minimum(wstart, NUM_NODES - 256), 16)
            shift = wstart - off

            d1 = pltpu.async_copy(b_hbm.at[pl.ds(off, 256)], bwin, sem)
            d2 = pltpu.async_copy(g_hbm.at[pl.ds(off, 256)], gwin, sem)
            d1.wait()
            d2.wait()

            def win_step(t, carry2):
                cnt, ended = carry2
                bv = bwin[pl.ds(shift + t * 16, 16)]
                gv = gwin[pl.ds(shift + t * 16, 16)]
                pidx = wstart + t * 16 + iota
                after = pidx >= s
                inseg = after & (bv == b)
                take = inseg & (gv == 1)
                pc = plsc.cumsum(take.astype(jnp.int32))
                sel = take & ((cnt + pc) <= IMAX)
                plsc.store_compressed(gtab.at[pl.ds(cnt, 16)], pidx, mask=sel)
                cnt = cnt + jnp.sum(sel.astype(jnp.int32))
                ended = jnp.maximum(
                    ended, jnp.max((after & (bv != b)).astype(jnp.int32)))
                return cnt, ended

            nst = (jnp.int32(256) - shift) >> 4
            cnt, ended = lax.fori_loop(0, nst, win_step, (cnt, ended))
            ended = jnp.maximum(
                ended, (wstart + 256 >= NUM_NODES).astype(jnp.int32))
            return cnt, ended, wstart + 256

        def win_cond(carry):
            cnt, ended, _ = carry
            return (cnt < IMAX) & (ended == 0)

        cc, _, _ = lax.while_loop(win_cond, win_body, (jnp.int32(0), jnp.int32(0), w0))

        tpairs = jnp.minimum(cc * (cc - 1) // 2, MAX_PAIRS)
        for h in range(2):
            kvec = h * 16 + iota
            kmask = kvec < MAX_PAIRS

            def ik_step(ii, ik):
                off = ii * cc - ii * (ii + 1) // 2
                term = (kvec >= off) & (ii <= cc - 2)
                return ik + term.astype(jnp.int32)

            i_k = lax.fori_loop(1, MAX_PAIRS, ik_step, jnp.zeros((16,), jnp.int32))
            offi = i_k * cc - i_k * (i_k + 1) // 2
            j_k = jnp.clip(kvec - offi + i_k + 1, 0, GTAB - 1)
            hp = (kvec < tpairs).astype(jnp.int32)
            gi = plsc.load_gather(gtab, [i_k])
            gj = plsc.load_gather(gtab, [j_k])
            bg = jnp.zeros((16,), jnp.int32) + b
            off0 = bl * MAX_PAIRS + h * 16
            if h == 0:
                idxi[pl.ds(off0, 16)] = gi
                idxj[pl.ds(off0, 16)] = gj
                idxg[pl.ds(off0, 16)] = bg
                hpb[pl.ds(off0, 16)] = hp
            else:
                plsc.store_compressed(idxi.at[pl.ds(off0, 16)], gi, mask=kmask)
                plsc.store_compressed(idxj.at[pl.ds(off0, 16)], gj, mask=kmask)
                plsc.store_compressed(idxg.at[pl.ds(off0, 16)], bg, mask=kmask)
                plsc.store_compressed(hpb.at[pl.ds(off0, 16)], hp, mask=kmask)

    obase = wid * PPW
    jobs = []
    r = 0
    for tab, idxb, outh in ((nf_hbm, idxi, fi_hbm), (nf_hbm, idxj, fj_hbm),
                            (gf_hbm, idxg, fg_hbm)):
        for t in range(NGC):
            jobs.append((tab, idxb, outh, t, r))
            r += 1
    descs = [pltpu.async_copy(tab.at[idxb.at[pl.ds(t * GCH, GCH)]],
                              rows.at[r], sem)
             for tab, idxb, outh, t, r in jobs]
    for d in descs:
        d.wait()
    outs = [pltpu.async_copy(rows.at[r], outh.at[pl.ds(obase + t * GCH, GCH)],
                             sem)
            for tab, idxb, outh, t, r in jobs]
    for d in outs:
        d.wait()
    pltpu.sync_copy(hpb.at[pl.ds(0, PPW)], hp_hbm.at[pl.ds(obase, PPW)])


_sc_gather = functools.partial(
    pl.kernel,
    out_type=(
        jax.ShapeDtypeStruct((NROWS, NODE_DIM), jnp.float32),
        jax.ShapeDtypeStruct((NROWS, NODE_DIM), jnp.float32),
        jax.ShapeDtypeStruct((NROWS, GLOBAL_DIM), jnp.float32),
        jax.ShapeDtypeStruct((NROWS,), jnp.int32),
    ),
    mesh=plsc.VectorSubcoreMesh(core_axis_name="c", subcore_axis_name="s",
                                num_cores=NC),
    scratch_types=(
        pltpu.VMEM((CHUNK + 16,), jnp.int32),
        pltpu.VMEM((STAB,), jnp.int32),
        pltpu.VMEM((NS * STAB,), jnp.int32),
        pltpu.VMEM((WL,), jnp.int32),
        pltpu.VMEM((256,), jnp.int32),
        pltpu.VMEM((256,), jnp.int32),
        pltpu.VMEM((GTAB,), jnp.int32),
        pltpu.VMEM((PPW + 8,), jnp.int32),
        pltpu.VMEM((PPW + 8,), jnp.int32),
        pltpu.VMEM((PPW + 8,), jnp.int32),
        pltpu.VMEM((PPW + 8,), jnp.int32),
        pltpu.VMEM((3 * NGC, GCH, NODE_DIM), jnp.float32),
        pltpu.VMEM_SHARED((NS * STAB,), jnp.int32),
        pltpu.SemaphoreType.DMA,
    ),
    compiler_params=pltpu.CompilerParams(needs_layout_passes=False),
)(_sc_body)


def _mlp_body(fi, fj, fg, w1a, w1b, w1g, b1, w2, b2, w3, b3, hp, vm, out):
    bf = jnp.bfloat16
    x = jnp.dot(fi[...].astype(bf), w1a[...].astype(bf),
                preferred_element_type=jnp.float32)
    x = x + jnp.dot(fj[...].astype(bf), w1b[...].astype(bf),
                    preferred_element_type=jnp.float32)
    x = x + jnp.dot(fg[...].astype(bf), w1g[...].astype(bf),
                    preferred_element_type=jnp.float32)
    h1 = jnp.maximum(x + b1[...], 0.0)
    h2 = jnp.maximum(
        jnp.dot(h1, w2[...], preferred_element_type=jnp.float32) + b2[...], 0.0)
    s = jnp.dot(h2, w3[...], preferred_element_type=jnp.float32) + b3[...]
    ok = (hp[...] > 0) & (vm[...] > 0)
    out[...] = jnp.where(ok, s, jnp.float32(NEG))


def kernel(node_features, global_features, group_mask, docking_valid_mask,
           batch, W1, b1, W2, b2, W3, b3):
    fi, fj, fg, hp = _sc_gather(batch.astype(jnp.int32),
                                group_mask.astype(jnp.int32),
                                node_features, global_features)

    w1a = W1[:NODE_DIM]
    w1b = W1[NODE_DIM:2 * NODE_DIM]
    w1g = W1[2 * NODE_DIM:]
    vm = docking_valid_mask.reshape(NROWS, 1).astype(jnp.int32)

    out = pl.pallas_call(
        _mlp_body,
        out_shape=jax.ShapeDtypeStruct((NROWS, 1), jnp.float32),
    )(fi, fj, fg, w1a, w1b, w1g, b1.reshape(1, -1), W2, b2.reshape(1, -1),
      W3, b3.reshape(1, 1), hp.reshape(NROWS, 1), vm)

    return out.reshape(BSZ, MAX_PAIRS)
